# baseline (speedup 1.0000x reference)
import functools

import jax
import jax.numpy as jnp
from jax import lax
from jax.experimental import pallas as pl
from jax.experimental.pallas import tpu as pltpu
from jax.experimental.pallas import tpu_sc as plsc

_NC = 2
_NS = 16
_NW = _NC * _NS
_K = 80
_NBUF = 2
_NSEC = 3
_GPS = 21
_SEC = _NBUF * _GPS
_CPW = _NSEC * _SEC
_EPW = _CPW * _K


def _make_sc_agg(N, D):
    ZTILES = 10
    RPT = N // ZTILES

    mesh = plsc.VectorSubcoreMesh(core_axis_name="c", subcore_axis_name="s")

    @functools.partial(
        pl.kernel,
        out_type=jax.ShapeDtypeStruct((_NC, N, D), jnp.float32),
        mesh=mesh,
        scratch_types=[
            pltpu.VMEM_SHARED((N + 8, D), jnp.float32),
            pltpu.VMEM((_EPW,), jnp.int32),
            pltpu.VMEM((2, _SEC, _K), jnp.int32),
        ]
        + [pltpu.VMEM((_K, D), jnp.float32)] * _NBUF
        + [pltpu.SemaphoreType.DMA] * (2 * _NBUF + 1),
    )
    def agg(h_hbm, src_hbm, dst_hbm, zeros_hbm, out_hbm,
            acc, src_v, dst_v, *bufs_and_sems):
        rows = bufs_and_sems[:_NBUF]
        gsem = bufs_and_sems[_NBUF:2 * _NBUF]
        ssem = bufs_and_sems[2 * _NBUF:3 * _NBUF]
        isem = bufs_and_sems[3 * _NBUF]
        c = lax.axis_index("c")
        s = lax.axis_index("s")
        wid = s * _NC + c

        @pl.when(s < ZTILES)
        def _zero():
            base = pl.multiple_of(s * RPT, 8)
            pltpu.sync_copy(zeros_hbm, acc.at[pl.ds(base, RPT)])

        pltpu.sync_copy(src_hbm.at[wid], src_v)
        pltpu.sync_copy(dst_hbm.at[wid, 0], dst_v.at[0])
        plsc.subcore_barrier()

        def _src_slice(j):
            return src_v.at[pl.ds(pl.multiple_of(j * _K, 8), _K)]

        def g_start(j, b):
            pltpu.async_copy(h_hbm.at[_src_slice(j)], rows[b], gsem[b])

        def g_wait(j, b):
            pltpu.make_async_copy(h_hbm.at[_src_slice(j)], rows[b],
                                  gsem[b]).wait()

        def s_start(dref, b):
            pltpu.async_copy(rows[b], acc.at[dref], ssem[b], add=True)

        def s_wait(dref, b):
            pltpu.make_async_copy(rows[b], acc.at[dref], ssem[b]).wait()

        for b in range(_NBUF):
            g_start(b, b)

        def body(g, carry):
            sec = lax.div(g, _GPS)
            i = lax.rem(g, _GPS)
            par = lax.rem(sec, 2)
            j = g * _NBUF

            @pl.when(jnp.logical_and(i == 0, sec > 0))
            def _wait_section():
                pltpu.make_async_copy(dst_hbm.at[wid, sec], dst_v.at[par],
                                      isem).wait()

            @pl.when(jnp.logical_and(i == 0, sec < _NSEC - 1))
            def _prefetch_section():
                pltpu.async_copy(dst_hbm.at[wid, sec + 1],
                                 dst_v.at[1 - par], isem)

            for b in range(_NBUF):
                g_wait(j + b, b)
                s_start(dst_v.at[par, i * _NBUF + b], b)
            for b in range(_NBUF):
                jb = j + b

                @pl.when(jb + _NBUF < _CPW)
                def _refill(jb=jb, b=b, par=par, i=i):
                    s_wait(dst_v.at[par, i * _NBUF + b], b)
                    g_start(jb + _NBUF, b)

            return carry

        lax.fori_loop(0, _NSEC * _GPS, body, 0)

        for b in range(_NBUF):
            s_wait(dst_v.at[(_NSEC - 1) % 2, _SEC - _NBUF + b], b)
        plsc.subcore_barrier()

        @pl.when(s < ZTILES)
        def _writeback():
            base = pl.multiple_of(s * RPT, 8)
            pltpu.sync_copy(acc.at[pl.ds(base, RPT)],
                            out_hbm.at[c, pl.ds(base, RPT)])

    return agg


def _dense_body(G, scale_ref, batch_ref, h_ref, p0_ref, p1_ref,
                w1_ref, b1_ref, g1_ref, bb1_ref,
                w2_ref, b2_ref, g2_ref, bb2_ref,
                h_out_ref, pool_ref):
    n = h_ref.shape[0]
    y = h_ref[...] * scale_ref[0, 0] + (p0_ref[...] + p1_ref[...])
    z = jnp.dot(y, w1_ref[...], preferred_element_type=jnp.float32) + b1_ref[...]
    mu = jnp.mean(z, axis=0, keepdims=True)
    var = jnp.mean(jnp.square(z - mu), axis=0, keepdims=True)
    z = g1_ref[...] * (z - mu) / jnp.sqrt(var + 1e-5) + bb1_ref[...]
    z = jnp.maximum(z, 0.0)
    z = jnp.dot(z, w2_ref[...], preferred_element_type=jnp.float32) + b2_ref[...]
    mu2 = jnp.mean(z, axis=0, keepdims=True)
    var2 = jnp.mean(jnp.square(z - mu2), axis=0, keepdims=True)
    h2 = g2_ref[...] * (z - mu2) / jnp.sqrt(var2 + 1e-5) + bb2_ref[...]
    h2 = jnp.maximum(h2, 0.0)
    h_out_ref[...] = h2
    oh = (batch_ref[...] == lax.broadcasted_iota(jnp.int32, (G, n), 0))
    pool_ref[...] = jnp.dot(oh.astype(jnp.float32), h2,
                            preferred_element_type=jnp.float32)


def _dense_layer(G, scale, batch2, h, p0, p1, prm):
    n, _ = h.shape
    hdim = prm["W1"].shape[1]
    return pl.pallas_call(
        functools.partial(_dense_body, G),
        out_shape=[
            jax.ShapeDtypeStruct((n, hdim), jnp.float32),
            jax.ShapeDtypeStruct((G, hdim), jnp.float32),
        ],
        in_specs=[pl.BlockSpec(memory_space=pltpu.SMEM)]
        + [pl.BlockSpec(memory_space=pltpu.VMEM)] * 12,
    )(scale, batch2, h, p0, p1,
      prm["W1"], prm["b1"].reshape(1, -1), prm["bn1_g"].reshape(1, -1),
      prm["bn1_b"].reshape(1, -1),
      prm["W2"], prm["b2"].reshape(1, -1), prm["bn_g"].reshape(1, -1),
      prm["bn_b"].reshape(1, -1))


def kernel(x, edge_index, batch, params):
    N, D = x.shape
    E = edge_index.shape[1]
    G = 64

    ei = edge_index.astype(jnp.int32)
    rpw = E // _NW
    ppw = _EPW - rpw
    pad_src = jnp.zeros((_NW, ppw), jnp.int32)
    pad_dst = jnp.broadcast_to(
        N + (jnp.arange(ppw, dtype=jnp.int32) % 8), (_NW, ppw))
    src = jnp.concatenate([ei[0].reshape(_NW, rpw), pad_src], axis=1)
    dst = jnp.concatenate([ei[1].reshape(_NW, rpw), pad_dst], axis=1)
    src = src.reshape(_NW, _EPW)
    dst = dst.reshape(_NW, _NSEC, _SEC, _K)
    zeros = jnp.zeros((N // 10, D), jnp.float32)
    batch2 = batch.astype(jnp.int32).reshape(1, N)

    sc_agg = _make_sc_agg(N, D)

    h = x
    reps, pools = [], []
    for prm in params:
        parts = sc_agg(h, src, dst, zeros)
        scale = jnp.reshape(1.0 + prm["eps"], (1, 1))
        h, pooled = _dense_layer(G, scale, batch2, h, parts[0], parts[1], prm)
        reps.append(h)
        pools.append(pooled)

    graph_rep = jnp.concatenate(pools, axis=1)
    node_rep = jnp.concatenate(reps, axis=1)
    return (graph_rep, node_rep)


# speedup vs baseline: 1.0028x; 1.0028x over previous
import functools

import jax
import jax.numpy as jnp
from jax import lax
from jax.experimental import pallas as pl
from jax.experimental.pallas import tpu as pltpu
from jax.experimental.pallas import tpu_sc as plsc

_NC = 2
_NS = 16
_NW = _NC * _NS
_K = 80
_NBUF = 2
_CPW = 126
_EPW = _CPW * _K


def _make_sc_agg(N, D):
    ZTILES = 10
    RPT = N // ZTILES

    mesh = plsc.VectorSubcoreMesh(core_axis_name="c", subcore_axis_name="s")

    @functools.partial(
        pl.kernel,
        out_type=jax.ShapeDtypeStruct((_NC, N, D), jnp.float32),
        mesh=mesh,
        scratch_types=[
            pltpu.VMEM_SHARED((N + 8, D), jnp.float32),
            pltpu.VMEM((_EPW,), jnp.int32),
            pltpu.VMEM((_CPW, _K), jnp.int32),
        ]
        + [pltpu.VMEM((_K, D), jnp.float32)] * _NBUF
        + [pltpu.SemaphoreType.DMA] * (2 * _NBUF),
    )
    def agg(h_hbm, src_hbm, dst_hbm, zeros_hbm, out_hbm,
            acc, src_v, dst_v, *bufs_and_sems):
        rows = bufs_and_sems[:_NBUF]
        gsem = bufs_and_sems[_NBUF:2 * _NBUF]
        ssem = bufs_and_sems[2 * _NBUF:3 * _NBUF]
        c = lax.axis_index("c")
        s = lax.axis_index("s")
        wid = s * _NC + c

        @pl.when(s < ZTILES)
        def _zero():
            base = pl.multiple_of(s * RPT, 8)
            pltpu.sync_copy(zeros_hbm, acc.at[pl.ds(base, RPT)])

        pltpu.sync_copy(src_hbm.at[wid], src_v)
        pltpu.sync_copy(dst_hbm.at[wid], dst_v)
        plsc.subcore_barrier()

        def _src_slice(j):
            return src_v.at[pl.ds(pl.multiple_of(j * _K, 8), _K)]

        def g_start(j, b):
            pltpu.async_copy(h_hbm.at[_src_slice(j)], rows[b], gsem[b])

        def g_wait(j, b):
            pltpu.make_async_copy(h_hbm.at[_src_slice(j)], rows[b],
                                  gsem[b]).wait()

        def s_start(dref, b):
            pltpu.async_copy(rows[b], acc.at[dref], ssem[b], add=True)

        def s_wait(dref, b):
            pltpu.make_async_copy(rows[b], acc.at[dref], ssem[b]).wait()

        for b in range(_NBUF):
            g_start(b, b)

        def body(g, carry):
            j = g * _NBUF
            for b in range(_NBUF):
                g_wait(j + b, b)
                s_start(dst_v.at[j + b], b)
            for b in range(_NBUF):
                jb = j + b

                @pl.when(jb + _NBUF < _CPW)
                def _refill(jb=jb, b=b):
                    s_wait(dst_v.at[jb], b)
                    g_start(jb + _NBUF, b)

            return carry

        lax.fori_loop(0, _CPW // _NBUF, body, 0)

        for b in range(_NBUF):
            s_wait(dst_v.at[_CPW - _NBUF + b], b)
        plsc.subcore_barrier()

        @pl.when(s < ZTILES)
        def _writeback():
            base = pl.multiple_of(s * RPT, 8)
            pltpu.sync_copy(acc.at[pl.ds(base, RPT)],
                            out_hbm.at[c, pl.ds(base, RPT)])

    return agg


def _dense_body(G, scale_ref, batch_ref, h_ref, p0_ref, p1_ref,
                w1_ref, b1_ref, g1_ref, bb1_ref,
                w2_ref, b2_ref, g2_ref, bb2_ref,
                h_out_ref, pool_ref):
    n = h_ref.shape[0]
    y = h_ref[...] * scale_ref[0, 0] + (p0_ref[...] + p1_ref[...])
    z = jnp.dot(y, w1_ref[...], preferred_element_type=jnp.float32) + b1_ref[...]
    mu = jnp.mean(z, axis=0, keepdims=True)
    var = jnp.mean(jnp.square(z - mu), axis=0, keepdims=True)
    z = g1_ref[...] * (z - mu) / jnp.sqrt(var + 1e-5) + bb1_ref[...]
    z = jnp.maximum(z, 0.0)
    z = jnp.dot(z, w2_ref[...], preferred_element_type=jnp.float32) + b2_ref[...]
    mu2 = jnp.mean(z, axis=0, keepdims=True)
    var2 = jnp.mean(jnp.square(z - mu2), axis=0, keepdims=True)
    h2 = g2_ref[...] * (z - mu2) / jnp.sqrt(var2 + 1e-5) + bb2_ref[...]
    h2 = jnp.maximum(h2, 0.0)
    h_out_ref[...] = h2
    oh = (batch_ref[...] == lax.broadcasted_iota(jnp.int32, (G, n), 0))
    pool_ref[...] = jnp.dot(oh.astype(jnp.float32), h2,
                            preferred_element_type=jnp.float32)


def _dense_layer(G, scale, batch2, h, p0, p1, prm):
    n, _ = h.shape
    hdim = prm["W1"].shape[1]
    return pl.pallas_call(
        functools.partial(_dense_body, G),
        out_shape=[
            jax.ShapeDtypeStruct((n, hdim), jnp.float32),
            jax.ShapeDtypeStruct((G, hdim), jnp.float32),
        ],
        in_specs=[pl.BlockSpec(memory_space=pltpu.SMEM)]
        + [pl.BlockSpec(memory_space=pltpu.VMEM)] * 12,
    )(scale, batch2, h, p0, p1,
      prm["W1"], prm["b1"].reshape(1, -1), prm["bn1_g"].reshape(1, -1),
      prm["bn1_b"].reshape(1, -1),
      prm["W2"], prm["b2"].reshape(1, -1), prm["bn_g"].reshape(1, -1),
      prm["bn_b"].reshape(1, -1))


def kernel(x, edge_index, batch, params):
    N, D = x.shape
    E = edge_index.shape[1]
    G = 64

    ei = edge_index.astype(jnp.int32)
    rpw = E // _NW
    ppw = _EPW - rpw
    pad_src = jnp.zeros((_NW, ppw), jnp.int32)
    pad_dst = jnp.broadcast_to(
        N + (jnp.arange(ppw, dtype=jnp.int32) % 8), (_NW, ppw))
    src = jnp.concatenate([ei[0].reshape(_NW, rpw), pad_src], axis=1)
    dst = jnp.concatenate([ei[1].reshape(_NW, rpw), pad_dst], axis=1)
    src = src.reshape(_NW, _EPW)
    dst = dst.reshape(_NW, _CPW, _K)
    zeros = jnp.zeros((N // 10, D), jnp.float32)
    batch2 = batch.astype(jnp.int32).reshape(1, N)

    sc_agg = _make_sc_agg(N, D)

    h = x
    reps, pools = [], []
    for prm in params:
        parts = sc_agg(h, src, dst, zeros)
        scale = jnp.reshape(1.0 + prm["eps"], (1, 1))
        h, pooled = _dense_layer(G, scale, batch2, h, parts[0], parts[1], prm)
        reps.append(h)
        pools.append(pooled)

    graph_rep = jnp.concatenate(pools, axis=1)
    node_rep = jnp.concatenate(reps, axis=1)
    return (graph_rep, node_rep)


# speedup vs baseline: 1.3684x; 1.3645x over previous
import functools

import jax
import jax.numpy as jnp
from jax import lax
from jax.experimental import pallas as pl
from jax.experimental.pallas import tpu as pltpu
from jax.experimental.pallas import tpu_sc as plsc

_NC = 2
_NS = 16
_NW = _NC * _NS
_K = 80
_NBUF = 2
_CPW = 126
_EPW = _CPW * _K


def _make_sc_agg(N, D):
    ZTILES = 10
    RPT = N // ZTILES

    mesh = plsc.VectorSubcoreMesh(core_axis_name="c", subcore_axis_name="s")

    @functools.partial(
        pl.kernel,
        out_type=jax.ShapeDtypeStruct((_NC, N, D), jnp.float32),
        mesh=mesh,
        scratch_types=[
            pltpu.VMEM_SHARED((N, D), jnp.float32),
            pltpu.VMEM((_EPW,), jnp.int32),
            pltpu.VMEM((_CPW, _K), jnp.int32),
        ]
        + [pltpu.VMEM((_K, D), jnp.float32)] * _NBUF
        + [pltpu.SemaphoreType.DMA] * (2 * _NBUF),
    )
    def agg(h_hbm, src_hbm, dst_hbm, zeros_hbm, out_hbm,
            acc, src_v, dst_v, *bufs_and_sems):
        rows = bufs_and_sems[:_NBUF]
        gsem = bufs_and_sems[_NBUF:2 * _NBUF]
        ssem = bufs_and_sems[2 * _NBUF:3 * _NBUF]
        c = lax.axis_index("c")
        s = lax.axis_index("s")
        wid = s * _NC + c

        @pl.when(s < ZTILES)
        def _zero():
            base = pl.multiple_of(s * RPT, 8)
            pltpu.sync_copy(zeros_hbm, acc.at[pl.ds(base, RPT)])

        pltpu.sync_copy(src_hbm.at[wid], src_v)
        pltpu.sync_copy(dst_hbm.at[wid], dst_v)
        plsc.subcore_barrier()

        def _src_slice(j):
            return src_v.at[pl.ds(pl.multiple_of(j * _K, 8), _K)]

        def g_start(j, b):
            pltpu.async_copy(h_hbm.at[_src_slice(j)], rows[b], gsem[b])

        def g_wait(j, b):
            pltpu.make_async_copy(h_hbm.at[_src_slice(j)], rows[b],
                                  gsem[b]).wait()

        def s_start(dref, b):
            pltpu.async_copy(rows[b], acc.at[dref], ssem[b], add=True)

        def s_wait(dref, b):
            pltpu.make_async_copy(rows[b], acc.at[dref], ssem[b]).wait()

        for b in range(_NBUF):
            g_start(b, b)

        def body(g, carry):
            j = g * _NBUF
            for b in range(_NBUF):
                g_wait(j + b, b)
                s_start(dst_v.at[j + b], b)
            for b in range(_NBUF):
                jb = j + b

                @pl.when(jb + _NBUF < _CPW)
                def _refill(jb=jb, b=b):
                    s_wait(dst_v.at[jb], b)
                    g_start(jb + _NBUF, b)

            return carry

        lax.fori_loop(0, _CPW // _NBUF, body, 0)

        for b in range(_NBUF):
            s_wait(dst_v.at[_CPW - _NBUF + b], b)
        plsc.subcore_barrier()

        @pl.when(s < ZTILES)
        def _writeback():
            base = pl.multiple_of(s * RPT, 8)
            pltpu.sync_copy(acc.at[pl.ds(base, RPT)],
                            out_hbm.at[c, pl.ds(base, RPT)])

    return agg


def _dense_body(G, scale_ref, batch_ref, h_ref, p0_ref, p1_ref,
                w1_ref, b1_ref, g1_ref, bb1_ref,
                w2_ref, b2_ref, g2_ref, bb2_ref,
                h_out_ref, pool_ref):
    n = h_ref.shape[0]
    y = h_ref[...] * scale_ref[0, 0] + (p0_ref[...] + p1_ref[...])
    z = jnp.dot(y, w1_ref[...], preferred_element_type=jnp.float32) + b1_ref[...]
    mu = jnp.mean(z, axis=0, keepdims=True)
    var = jnp.mean(jnp.square(z - mu), axis=0, keepdims=True)
    z = g1_ref[...] * (z - mu) / jnp.sqrt(var + 1e-5) + bb1_ref[...]
    z = jnp.maximum(z, 0.0)
    z = jnp.dot(z, w2_ref[...], preferred_element_type=jnp.float32) + b2_ref[...]
    mu2 = jnp.mean(z, axis=0, keepdims=True)
    var2 = jnp.mean(jnp.square(z - mu2), axis=0, keepdims=True)
    h2 = g2_ref[...] * (z - mu2) / jnp.sqrt(var2 + 1e-5) + bb2_ref[...]
    h2 = jnp.maximum(h2, 0.0)
    h_out_ref[...] = h2
    oh = (batch_ref[...] == lax.broadcasted_iota(jnp.int32, (G, n), 0))
    pool_ref[...] = jnp.dot(oh.astype(jnp.float32), h2,
                            preferred_element_type=jnp.float32)


def _dense_layer(G, scale, batch2, h, p0, p1, prm):
    n, _ = h.shape
    hdim = prm["W1"].shape[1]
    return pl.pallas_call(
        functools.partial(_dense_body, G),
        out_shape=[
            jax.ShapeDtypeStruct((n, hdim), jnp.float32),
            jax.ShapeDtypeStruct((G, hdim), jnp.float32),
        ],
        in_specs=[pl.BlockSpec(memory_space=pltpu.SMEM)]
        + [pl.BlockSpec(memory_space=pltpu.VMEM)] * 12,
    )(scale, batch2, h, p0, p1,
      prm["W1"], prm["b1"].reshape(1, -1), prm["bn1_g"].reshape(1, -1),
      prm["bn1_b"].reshape(1, -1),
      prm["W2"], prm["b2"].reshape(1, -1), prm["bn_g"].reshape(1, -1),
      prm["bn_b"].reshape(1, -1))


def kernel(x, edge_index, batch, params):
    N, D = x.shape
    E = edge_index.shape[1]
    G = 64

    ei = edge_index.astype(jnp.int32)
    rpw = E // _NW
    ppw = _EPW - rpw
    pad_src = jnp.broadcast_to(
        N + (jnp.arange(ppw, dtype=jnp.int32) % 8), (_NW, ppw))
    pad_dst = (jnp.arange(_NW, dtype=jnp.int32)[:, None] * ppw
               + jnp.arange(ppw, dtype=jnp.int32)[None, :])
    src = jnp.concatenate([ei[0].reshape(_NW, rpw), pad_src], axis=1)
    dst = jnp.concatenate([ei[1].reshape(_NW, rpw), pad_dst], axis=1)
    src = src.reshape(_NW, _EPW)
    dst = dst.reshape(_NW, _CPW, _K)
    zrows = jnp.zeros((8, D), jnp.float32)
    zeros = jnp.zeros((N // 10, D), jnp.float32)
    batch2 = batch.astype(jnp.int32).reshape(1, N)

    sc_agg = _make_sc_agg(N, D)

    h = x
    reps, pools = [], []
    for prm in params:
        h_pad = jnp.concatenate([h, zrows], axis=0)
        parts = sc_agg(h_pad, src, dst, zeros)
        scale = jnp.reshape(1.0 + prm["eps"], (1, 1))
        h, pooled = _dense_layer(G, scale, batch2, h, parts[0], parts[1], prm)
        reps.append(h)
        pools.append(pooled)

    graph_rep = jnp.concatenate(pools, axis=1)
    node_rep = jnp.concatenate(reps, axis=1)
    return (graph_rep, node_rep)


# speedup vs baseline: 1.6619x; 1.2145x over previous
import functools

import jax
import jax.numpy as jnp
from jax import lax
from jax.experimental import pallas as pl
from jax.experimental.pallas import tpu as pltpu
from jax.experimental.pallas import tpu_sc as plsc

_NC = 2
_NS = 16
_NW = _NC * _NS
_K = 80
_NBUF = 3
_NSEC = 6
_GPS = 7
_SEC = _NBUF * _GPS
_CPW = _NSEC * _SEC
_EPW = _CPW * _K


def _make_sc_agg(N, D):
    ZTILES = 10
    RPT = N // ZTILES

    mesh = plsc.VectorSubcoreMesh(core_axis_name="c", subcore_axis_name="s")

    @functools.partial(
        pl.kernel,
        out_type=jax.ShapeDtypeStruct((_NC, N, D), jnp.float32),
        mesh=mesh,
        scratch_types=[
            pltpu.VMEM_SHARED((N, D), jnp.float32),
            pltpu.VMEM((_EPW,), jnp.int32),
            pltpu.VMEM((2, _SEC, _K), jnp.int32),
        ]
        + [pltpu.VMEM((_K, D), jnp.float32)] * _NBUF
        + [pltpu.SemaphoreType.DMA] * (2 * _NBUF + 1),
    )
    def agg(h_hbm, src_hbm, dst_hbm, zeros_hbm, out_hbm,
            acc, src_v, dst_v, *bufs_and_sems):
        rows = bufs_and_sems[:_NBUF]
        gsem = bufs_and_sems[_NBUF:2 * _NBUF]
        ssem = bufs_and_sems[2 * _NBUF:3 * _NBUF]
        isem = bufs_and_sems[3 * _NBUF]
        c = lax.axis_index("c")
        s = lax.axis_index("s")
        wid = s * _NC + c

        @pl.when(s < ZTILES)
        def _zero():
            base = pl.multiple_of(s * RPT, 8)
            pltpu.sync_copy(zeros_hbm, acc.at[pl.ds(base, RPT)])

        pltpu.sync_copy(src_hbm.at[wid], src_v)
        pltpu.sync_copy(dst_hbm.at[wid, 0], dst_v.at[0])
        plsc.subcore_barrier()

        def _src_slice(j):
            return src_v.at[pl.ds(pl.multiple_of(j * _K, 8), _K)]

        def g_start(j, b):
            pltpu.async_copy(h_hbm.at[_src_slice(j)], rows[b], gsem[b])

        def g_wait(j, b):
            pltpu.make_async_copy(h_hbm.at[_src_slice(j)], rows[b],
                                  gsem[b]).wait()

        def s_start(dref, b):
            pltpu.async_copy(rows[b], acc.at[dref], ssem[b], add=True)

        def s_wait(dref, b):
            pltpu.make_async_copy(rows[b], acc.at[dref], ssem[b]).wait()

        for b in range(_NBUF):
            g_start(b, b)

        def body(g, carry):
            sec = lax.div(g, _GPS)
            i = lax.rem(g, _GPS)
            par = lax.rem(sec, 2)
            j = g * _NBUF

            @pl.when(jnp.logical_and(i == 0, sec > 0))
            def _wait_section():
                pltpu.make_async_copy(dst_hbm.at[wid, sec], dst_v.at[par],
                                      isem).wait()

            @pl.when(jnp.logical_and(i == 0, sec < _NSEC - 1))
            def _prefetch_section():
                pltpu.async_copy(dst_hbm.at[wid, sec + 1],
                                 dst_v.at[1 - par], isem)

            for b in range(_NBUF):
                g_wait(j + b, b)
                s_start(dst_v.at[par, i * _NBUF + b], b)
            for b in range(_NBUF):
                jb = j + b

                @pl.when(jb + _NBUF < _CPW)
                def _refill(jb=jb, b=b, par=par, i=i):
                    s_wait(dst_v.at[par, i * _NBUF + b], b)
                    g_start(jb + _NBUF, b)

            return carry

        lax.fori_loop(0, _NSEC * _GPS, body, 0)

        for b in range(_NBUF):
            s_wait(dst_v.at[(_NSEC - 1) % 2, _SEC - _NBUF + b], b)
        plsc.subcore_barrier()

        @pl.when(s < ZTILES)
        def _writeback():
            base = pl.multiple_of(s * RPT, 8)
            pltpu.sync_copy(acc.at[pl.ds(base, RPT)],
                            out_hbm.at[c, pl.ds(base, RPT)])

    return agg


def _dense_body(G, scale_ref, batch_ref, h_ref, p0_ref, p1_ref,
                w1_ref, b1_ref, g1_ref, bb1_ref,
                w2_ref, b2_ref, g2_ref, bb2_ref,
                h_out_ref, pool_ref):
    n = p0_ref.shape[0]
    y = h_ref[pl.ds(0, n)] * scale_ref[0, 0] + (p0_ref[...] + p1_ref[...])
    z = jnp.dot(y, w1_ref[...], preferred_element_type=jnp.float32) + b1_ref[...]
    mu = jnp.mean(z, axis=0, keepdims=True)
    var = jnp.mean(jnp.square(z - mu), axis=0, keepdims=True)
    z = g1_ref[...] * (z - mu) / jnp.sqrt(var + 1e-5) + bb1_ref[...]
    z = jnp.maximum(z, 0.0)
    z = jnp.dot(z, w2_ref[...], preferred_element_type=jnp.float32) + b2_ref[...]
    mu2 = jnp.mean(z, axis=0, keepdims=True)
    var2 = jnp.mean(jnp.square(z - mu2), axis=0, keepdims=True)
    h2 = g2_ref[...] * (z - mu2) / jnp.sqrt(var2 + 1e-5) + bb2_ref[...]
    h2 = jnp.maximum(h2, 0.0)
    h_out_ref[pl.ds(0, n)] = h2
    h_out_ref[pl.ds(n, 8)] = jnp.zeros((8, h2.shape[1]), jnp.float32)
    oh = (batch_ref[...] == lax.broadcasted_iota(jnp.int32, (G, n), 0))
    pool_ref[...] = jnp.dot(oh.astype(jnp.float32), h2,
                            preferred_element_type=jnp.float32)


def _dense_layer(G, scale, batch2, h, p0, p1, prm):
    n = p0.shape[0]
    hdim = prm["W1"].shape[1]
    return pl.pallas_call(
        functools.partial(_dense_body, G),
        out_shape=[
            jax.ShapeDtypeStruct((n + 8, hdim), jnp.float32),
            jax.ShapeDtypeStruct((G, hdim), jnp.float32),
        ],
        in_specs=[pl.BlockSpec(memory_space=pltpu.SMEM)]
        + [pl.BlockSpec(memory_space=pltpu.VMEM)] * 12,
    )(scale, batch2, h, p0, p1,
      prm["W1"], prm["b1"].reshape(1, -1), prm["bn1_g"].reshape(1, -1),
      prm["bn1_b"].reshape(1, -1),
      prm["W2"], prm["b2"].reshape(1, -1), prm["bn_g"].reshape(1, -1),
      prm["bn_b"].reshape(1, -1))


def kernel(x, edge_index, batch, params):
    N, D = x.shape
    E = edge_index.shape[1]
    G = 64

    ei = edge_index.astype(jnp.int32)
    rpw = E // _NW
    ppw = _EPW - rpw
    pad_src = jnp.broadcast_to(
        N + (jnp.arange(ppw, dtype=jnp.int32) % 8), (_NW, ppw))
    pad_dst = (jnp.arange(_NW, dtype=jnp.int32)[:, None] * ppw
               + jnp.arange(ppw, dtype=jnp.int32)[None, :])
    src = jnp.concatenate([ei[0].reshape(_NW, rpw), pad_src], axis=1)
    dst = jnp.concatenate([ei[1].reshape(_NW, rpw), pad_dst], axis=1)
    src = src.reshape(_NW, _EPW)
    dst = dst.reshape(_NW, _NSEC, _SEC, _K)
    zeros = jnp.zeros((N // 10, D), jnp.float32)
    batch2 = batch.astype(jnp.int32).reshape(1, N)

    sc_agg = _make_sc_agg(N, D)

    h = jnp.concatenate([x, jnp.zeros((8, D), jnp.float32)], axis=0)
    reps, pools = [], []
    for prm in params:
        parts = sc_agg(h, src, dst, zeros)
        scale = jnp.reshape(1.0 + prm["eps"], (1, 1))
        h, pooled = _dense_layer(G, scale, batch2, h, parts[0], parts[1], prm)
        reps.append(h[:N])
        pools.append(pooled)

    graph_rep = jnp.concatenate(pools, axis=1)
    node_rep = jnp.concatenate(reps, axis=1)
    return (graph_rep, node_rep)
